# baseline (speedup 1.0000x reference)
import functools

import jax
import jax.numpy as jnp
from jax import lax
from jax.experimental import pallas as pl
from jax.experimental.pallas import tpu as pltpu
from jax.experimental.pallas import tpu_sc as plsc

_LANES = 128
_NC = 2
_NS = 16
_NW = _NC * _NS
_CH = 12800
_R0_NUM = 5
_R0_DEN = 16


def _round_up(a, b):
    return (a + b - 1) // b * b


def _make_edge_pass(n_pad, e_pad, do_gather):
    rounds = e_pad // (_NS * _CH)
    r0 = rounds * _R0_NUM // _R0_DEN
    r1 = rounds - r0
    ts = n_pad // _NS
    mesh = plsc.VectorSubcoreMesh(core_axis_name="c", subcore_axis_name="s")

    out_type = jax.ShapeDtypeStruct((_NC, n_pad), jnp.float32)

    if do_gather:
        scratch = [
            pltpu.VMEM((_CH,), jnp.int32),
            pltpu.VMEM((_CH,), jnp.int32),
            pltpu.VMEM((_CH,), jnp.float32),
            pltpu.VMEM_SHARED((n_pad,), jnp.float32),
            pltpu.VMEM_SHARED((n_pad,), jnp.float32),
            pltpu.SemaphoreType.DMA,
        ]

        @functools.partial(pl.kernel, mesh=mesh, out_type=out_type,
                           scratch_types=scratch)
        def k(src_hbm, dst_hbm, table_hbm, zeros_hbm, out_hbm,
              src_v, dst_v, vals_v, acc_sh, tab_sh, sem):
            c = lax.axis_index("c")
            s = lax.axis_index("s")
            soff = pl.multiple_of(s * ts, 8)
            pltpu.sync_copy(table_hbm.at[pl.ds(soff, ts)],
                            tab_sh.at[pl.ds(soff, ts)])
            pltpu.sync_copy(zeros_hbm.at[pl.ds(soff, ts)],
                            acc_sh.at[pl.ds(soff, ts)])
            plsc.subcore_barrier()
            my_rounds = jnp.where(c == 0, jnp.int32(r0), jnp.int32(r1))
            cbase = jnp.where(c == 0, jnp.int32(0),
                              jnp.int32(r0)) * jnp.int32(_NS * _CH)
            tbase = cbase + s.astype(jnp.int32) * jnp.int32(_CH)

            def body(i, carry):
                eoff = pl.multiple_of(
                    tbase + i * jnp.int32(_NS * _CH), 8)
                pltpu.sync_copy(src_hbm.at[pl.ds(eoff, _CH)], src_v)
                pltpu.sync_copy(dst_hbm.at[pl.ds(eoff, _CH)], dst_v)
                pltpu.async_copy(tab_sh.at[src_v], vals_v, sem).wait()
                pltpu.sync_copy(vals_v, acc_sh.at[dst_v], add=True)
                return carry

            lax.fori_loop(jnp.int32(0), my_rounds, body, jnp.int32(0))
            plsc.subcore_barrier()
            pltpu.sync_copy(acc_sh.at[pl.ds(soff, ts)],
                            out_hbm.at[c, pl.ds(soff, ts)])
        return k
    else:
        scratch = [
            pltpu.VMEM((_CH,), jnp.int32),
            pltpu.VMEM((_CH,), jnp.float32),
            pltpu.VMEM_SHARED((n_pad,), jnp.float32),
            pltpu.SemaphoreType.DMA,
        ]

        @functools.partial(pl.kernel, mesh=mesh, out_type=out_type,
                           scratch_types=scratch)
        def k(dst_hbm, ones_hbm, zeros_hbm, out_hbm,
              dst_v, vals_v, acc_sh, sem):
            c = lax.axis_index("c")
            s = lax.axis_index("s")
            soff = pl.multiple_of(s * ts, 8)
            pltpu.sync_copy(zeros_hbm.at[pl.ds(soff, ts)],
                            acc_sh.at[pl.ds(soff, ts)])
            pltpu.sync_copy(ones_hbm, vals_v)
            plsc.subcore_barrier()
            my_rounds = jnp.where(c == 0, jnp.int32(r0), jnp.int32(r1))
            cbase = jnp.where(c == 0, jnp.int32(0),
                              jnp.int32(r0)) * jnp.int32(_NS * _CH)
            tbase = cbase + s.astype(jnp.int32) * jnp.int32(_CH)

            def body(i, carry):
                eoff = pl.multiple_of(
                    tbase + i * jnp.int32(_NS * _CH), 8)
                pltpu.sync_copy(dst_hbm.at[pl.ds(eoff, _CH)], dst_v)
                pltpu.sync_copy(vals_v, acc_sh.at[dst_v], add=True)
                return carry

            lax.fori_loop(jnp.int32(0), my_rounds, body, jnp.int32(0))
            plsc.subcore_barrier()
            pltpu.sync_copy(acc_sh.at[pl.ds(soff, ts)],
                            out_hbm.at[c, pl.ds(soff, ts)])
        return k


def _tc_stage1(degp, x2):
    rn = x2.shape[0]

    def body(degp_ref, x_ref, dinv_ref, u_ref, invdeg_ref):
        deg = degp_ref[0] + degp_ref[1] + 1.0
        dinv = lax.rsqrt(deg)
        dinv_ref[...] = dinv
        invdeg_ref[...] = 1.0 / deg
        u_ref[...] = dinv * x_ref[...]

    shp = jax.ShapeDtypeStruct((rn, _LANES), jnp.float32)
    return pl.pallas_call(body, out_shape=[shp, shp, shp])(degp, x2)


def _tc_stage2(accp, dinv, x2, invdeg, aux):
    rn = x2.shape[0]

    def body(accp_ref, dinv_ref, x_ref, invdeg_ref, aux_ref, g_ref, v_ref):
        dinv = dinv_ref[...]
        s = dinv * (accp_ref[0] + accp_ref[1]) + x_ref[...] * invdeg_ref[...]
        g = jnp.zeros_like(s)
        for j in range(16):
            w1j = aux_ref[0, j]
            b1j = aux_ref[1, j]
            w2j = aux_ref[2, j]
            g = g + jnp.maximum(s * w1j + b1j, 0.0) * w2j
        g_ref[...] = g
        v_ref[...] = dinv * g

    shp = jax.ShapeDtypeStruct((rn, _LANES), jnp.float32)
    return pl.pallas_call(body, out_shape=[shp, shp])(
        accp, dinv, x2, invdeg, aux)


def _tc_stage3(accp, dinv, g, invdeg, aux):
    rn = g.shape[0]

    def body(accp_ref, dinv_ref, g_ref, invdeg_ref, aux_ref, out_ref):
        h = (dinv_ref[...] * (accp_ref[0] + accp_ref[1])
             + g_ref[...] * invdeg_ref[...] + aux_ref[3, 0])
        z = h - h
        lse = jnp.log(jnp.exp(z))
        out_ref[...] = z - lse

    shp = jax.ShapeDtypeStruct((rn, _LANES), jnp.float32)
    return pl.pallas_call(body, out_shape=shp)(accp, dinv, g, invdeg, aux)


def kernel(x, edge_index, W1, b1, W2, b2):
    n = x.shape[0]
    e = edge_index.shape[1]
    n_pad = _round_up(n + 1, _LANES * 8)
    rn = n_pad // _LANES
    e_pad = _round_up(e, _NS * _CH)
    dummy = n_pad - 1

    ei = edge_index.astype(jnp.int32)
    pad = jnp.full((e_pad - e,), dummy, jnp.int32)
    src1 = jnp.concatenate([ei[0], pad])
    dst1 = jnp.concatenate([ei[1], pad])

    x1 = jnp.zeros((n_pad,), jnp.float32).at[:n].set(x[:, 0].astype(jnp.float32))
    x2 = x1.reshape(rn, _LANES)
    zeros_n = jnp.zeros((n_pad,), jnp.float32)
    ones_ch = jnp.ones((_CH,), jnp.float32)

    aux = jnp.zeros((8, _LANES), jnp.float32)
    aux = aux.at[0, :16].set(W1[0].astype(jnp.float32))
    aux = aux.at[1, :16].set(b1.astype(jnp.float32))
    aux = aux.at[2, :16].set(W2[:, 0].astype(jnp.float32))
    aux = aux.at[3, 0].set(b2[0].astype(jnp.float32))

    count_k = _make_edge_pass(n_pad, e_pad, do_gather=False)
    gs_k = _make_edge_pass(n_pad, e_pad, do_gather=True)

    degp = count_k(dst1, ones_ch, zeros_n)
    dinv, u, invdeg = _tc_stage1(degp.reshape(_NC, rn, _LANES), x2)

    acc1 = gs_k(src1, dst1, u.reshape(n_pad), zeros_n)
    g, v = _tc_stage2(acc1.reshape(_NC, rn, _LANES), dinv, x2, invdeg, aux)

    acc2 = gs_k(src1, dst1, v.reshape(n_pad), zeros_n)
    lsm = _tc_stage3(acc2.reshape(_NC, rn, _LANES), dinv, g, invdeg, aux)

    return lsm.reshape(n_pad)[:n].reshape(n, 1).astype(jnp.float64)


# speedup vs baseline: 1.1956x; 1.1956x over previous
import functools

import jax
import jax.numpy as jnp
from jax import lax
from jax.experimental import pallas as pl
from jax.experimental.pallas import tpu as pltpu
from jax.experimental.pallas import tpu_sc as plsc

_LANES = 128
_NC = 2
_NS = 16
_NW = _NC * _NS
_CH = 12800
_R0_NUM = 11
_R0_DEN = 16


def _round_up(a, b):
    return (a + b - 1) // b * b


def _make_edge_pass(n_pad, e_pad, do_gather):
    rounds = e_pad // (_NS * _CH)
    r0 = rounds * _R0_NUM // _R0_DEN
    r1 = rounds - r0
    ts = n_pad // _NS
    mesh = plsc.VectorSubcoreMesh(core_axis_name="c", subcore_axis_name="s")

    out_type = jax.ShapeDtypeStruct((_NC, n_pad), jnp.float32)

    if do_gather:
        scratch = [
            pltpu.VMEM((_CH,), jnp.int32),
            pltpu.VMEM((_CH,), jnp.int32),
            pltpu.VMEM((_CH,), jnp.float32),
            pltpu.VMEM_SHARED((n_pad,), jnp.float32),
            pltpu.VMEM_SHARED((n_pad,), jnp.float32),
            pltpu.SemaphoreType.DMA,
        ]

        @functools.partial(pl.kernel, mesh=mesh, out_type=out_type,
                           scratch_types=scratch)
        def k(src_hbm, dst_hbm, table_hbm, zeros_hbm, out_hbm,
              src_v, dst_v, vals_v, acc_sh, tab_sh, sem):
            c = lax.axis_index("c")
            s = lax.axis_index("s")
            soff = pl.multiple_of(s * ts, 8)
            pltpu.sync_copy(table_hbm.at[pl.ds(soff, ts)],
                            tab_sh.at[pl.ds(soff, ts)])
            pltpu.sync_copy(zeros_hbm.at[pl.ds(soff, ts)],
                            acc_sh.at[pl.ds(soff, ts)])
            plsc.subcore_barrier()
            my_rounds = jnp.where(c == 0, jnp.int32(r0), jnp.int32(r1))
            cbase = jnp.where(c == 0, jnp.int32(0),
                              jnp.int32(r0)) * jnp.int32(_NS * _CH)
            tbase = cbase + s.astype(jnp.int32) * jnp.int32(_CH)

            def body(i, carry):
                eoff = pl.multiple_of(
                    tbase + i * jnp.int32(_NS * _CH), 8)
                pltpu.sync_copy(src_hbm.at[pl.ds(eoff, _CH)], src_v)
                pltpu.sync_copy(dst_hbm.at[pl.ds(eoff, _CH)], dst_v)
                pltpu.async_copy(tab_sh.at[src_v], vals_v, sem).wait()
                pltpu.sync_copy(vals_v, acc_sh.at[dst_v], add=True)
                return carry

            lax.fori_loop(jnp.int32(0), my_rounds, body, jnp.int32(0))
            plsc.subcore_barrier()
            pltpu.sync_copy(acc_sh.at[pl.ds(soff, ts)],
                            out_hbm.at[c, pl.ds(soff, ts)])
        return k
    else:
        scratch = [
            pltpu.VMEM((_CH,), jnp.int32),
            pltpu.VMEM((_CH,), jnp.float32),
            pltpu.VMEM_SHARED((n_pad,), jnp.float32),
            pltpu.SemaphoreType.DMA,
        ]

        @functools.partial(pl.kernel, mesh=mesh, out_type=out_type,
                           scratch_types=scratch)
        def k(dst_hbm, ones_hbm, zeros_hbm, out_hbm,
              dst_v, vals_v, acc_sh, sem):
            c = lax.axis_index("c")
            s = lax.axis_index("s")
            soff = pl.multiple_of(s * ts, 8)
            pltpu.sync_copy(zeros_hbm.at[pl.ds(soff, ts)],
                            acc_sh.at[pl.ds(soff, ts)])
            pltpu.sync_copy(ones_hbm, vals_v)
            plsc.subcore_barrier()
            my_rounds = jnp.where(c == 0, jnp.int32(r0), jnp.int32(r1))
            cbase = jnp.where(c == 0, jnp.int32(0),
                              jnp.int32(r0)) * jnp.int32(_NS * _CH)
            tbase = cbase + s.astype(jnp.int32) * jnp.int32(_CH)

            def body(i, carry):
                eoff = pl.multiple_of(
                    tbase + i * jnp.int32(_NS * _CH), 8)
                pltpu.sync_copy(dst_hbm.at[pl.ds(eoff, _CH)], dst_v)
                pltpu.sync_copy(vals_v, acc_sh.at[dst_v], add=True)
                return carry

            lax.fori_loop(jnp.int32(0), my_rounds, body, jnp.int32(0))
            plsc.subcore_barrier()
            pltpu.sync_copy(acc_sh.at[pl.ds(soff, ts)],
                            out_hbm.at[c, pl.ds(soff, ts)])
        return k


def _tc_stage1(degp, x2):
    rn = x2.shape[0]

    def body(degp_ref, x_ref, dinv_ref, u_ref, invdeg_ref):
        deg = degp_ref[0] + degp_ref[1] + 1.0
        dinv = lax.rsqrt(deg)
        dinv_ref[...] = dinv
        invdeg_ref[...] = 1.0 / deg
        u_ref[...] = dinv * x_ref[...]

    shp = jax.ShapeDtypeStruct((rn, _LANES), jnp.float32)
    return pl.pallas_call(body, out_shape=[shp, shp, shp])(degp, x2)


def _tc_stage2(accp, dinv, x2, invdeg, aux):
    rn = x2.shape[0]

    def body(accp_ref, dinv_ref, x_ref, invdeg_ref, aux_ref, g_ref, v_ref):
        dinv = dinv_ref[...]
        s = dinv * (accp_ref[0] + accp_ref[1]) + x_ref[...] * invdeg_ref[...]
        g = jnp.zeros_like(s)
        for j in range(16):
            w1j = aux_ref[0, j]
            b1j = aux_ref[1, j]
            w2j = aux_ref[2, j]
            g = g + jnp.maximum(s * w1j + b1j, 0.0) * w2j
        g_ref[...] = g
        v_ref[...] = dinv * g

    shp = jax.ShapeDtypeStruct((rn, _LANES), jnp.float32)
    return pl.pallas_call(body, out_shape=[shp, shp])(
        accp, dinv, x2, invdeg, aux)


def _tc_stage3(accp, dinv, g, invdeg, aux):
    rn = g.shape[0]

    def body(accp_ref, dinv_ref, g_ref, invdeg_ref, aux_ref, out_ref):
        h = (dinv_ref[...] * (accp_ref[0] + accp_ref[1])
             + g_ref[...] * invdeg_ref[...] + aux_ref[3, 0])
        z = h - h
        lse = jnp.log(jnp.exp(z))
        out_ref[...] = z - lse

    shp = jax.ShapeDtypeStruct((rn, _LANES), jnp.float32)
    return pl.pallas_call(body, out_shape=shp)(accp, dinv, g, invdeg, aux)


def kernel(x, edge_index, W1, b1, W2, b2):
    n = x.shape[0]
    e = edge_index.shape[1]
    n_pad = _round_up(n + 1, _LANES * 8)
    rn = n_pad // _LANES
    e_pad = _round_up(e, _NS * _CH)
    dummy = n_pad - 1

    ei = edge_index.astype(jnp.int32)
    pad = jnp.full((e_pad - e,), dummy, jnp.int32)
    src1 = jnp.concatenate([ei[0], pad])
    dst1 = jnp.concatenate([ei[1], pad])

    x1 = jnp.zeros((n_pad,), jnp.float32).at[:n].set(x[:, 0].astype(jnp.float32))
    x2 = x1.reshape(rn, _LANES)
    zeros_n = jnp.zeros((n_pad,), jnp.float32)
    ones_ch = jnp.ones((_CH,), jnp.float32)

    aux = jnp.zeros((8, _LANES), jnp.float32)
    aux = aux.at[0, :16].set(W1[0].astype(jnp.float32))
    aux = aux.at[1, :16].set(b1.astype(jnp.float32))
    aux = aux.at[2, :16].set(W2[:, 0].astype(jnp.float32))
    aux = aux.at[3, 0].set(b2[0].astype(jnp.float32))

    count_k = _make_edge_pass(n_pad, e_pad, do_gather=False)
    gs_k = _make_edge_pass(n_pad, e_pad, do_gather=True)

    degp = count_k(dst1, ones_ch, zeros_n)
    dinv, u, invdeg = _tc_stage1(degp.reshape(_NC, rn, _LANES), x2)

    acc1 = gs_k(src1, dst1, u.reshape(n_pad), zeros_n)
    g, v = _tc_stage2(acc1.reshape(_NC, rn, _LANES), dinv, x2, invdeg, aux)

    acc2 = gs_k(src1, dst1, v.reshape(n_pad), zeros_n)
    lsm = _tc_stage3(acc2.reshape(_NC, rn, _LANES), dinv, g, invdeg, aux)

    return lsm.reshape(n_pad)[:n].reshape(n, 1).astype(jnp.float64)
